# baseline (speedup 1.0000x reference)
import functools

import jax
import jax.numpy as jnp
import numpy as np
from jax.experimental import pallas as pl
from jax.experimental.pallas import tpu as pltpu
from jax.experimental.pallas import tpu_sc as plsc

B, S, D = 1, 2048, 1024
H, KV, HD = 16, 4, 64
E, K, DFF = 8, 2, 2048
C = 512
EC = E * C
CLIP = 8.0
ROPE = 500000.0
EPS = 1e-5
REP = H // KV
HALF = HD // 2

BSQ = 256
QKVW = D + 2 * KV * HD
FB = 512
SB = 512

_f32 = jnp.float32


def _roll_lanes(t, sh):
    return jnp.concatenate([t[:, sh:], t[:, :sh]], axis=1)


def _rope(t, cs, ss, nlanes, scale=1.0):
    within = jax.lax.broadcasted_iota(jnp.int32, (1, nlanes), 1) % HD
    reps = nlanes // HALF
    cosv = jnp.concatenate([cs] * reps, axis=1)
    sinv = jnp.concatenate([ss] * reps, axis=1)
    rot = jnp.where(within < HALF, -_roll_lanes(t, HALF), _roll_lanes(t, nlanes - HALF))
    return (t * cosv + rot * sinv) * _f32(scale)


def _prologue_kern(x_ref, pos_ref, g1_ref, wqkv_ref, q_ref, kt_ref, v3_ref):
    x = x_ref[...]
    mu = jnp.mean(x, axis=-1, keepdims=True)
    var = jnp.mean((x - mu) ** 2, axis=-1, keepdims=True)
    h = (x - mu) * jax.lax.rsqrt(var + EPS) * g1_ref[...]
    qkv = jnp.dot(h, wqkv_ref[...], preferred_element_type=_f32)
    qkv = jnp.clip(qkv, -CLIP, CLIP)
    pos_f = pos_ref[...].astype(_f32)
    j = jax.lax.broadcasted_iota(jnp.int32, (1, HALF), 1).astype(_f32)
    inv = jnp.exp(j * _f32(-np.log(ROPE) / HALF))
    theta = pos_f * inv
    cs = jnp.cos(theta)
    ss = jnp.sin(theta)
    q_ref[...] = _rope(qkv[:, :D], cs, ss, D, scale=1.0 / np.sqrt(HD))
    k = _rope(qkv[:, D:D + KV * HD], cs, ss, KV * HD)
    kt_ref[...] = k.T
    v = qkv[:, D + KV * HD:]
    v3_ref[...] = jnp.concatenate(
        [v[:, kh * HD:(kh + 1) * HD][None] for kh in range(KV)], axis=0)


BQ = 512
BK = 256
NQB = S // BQ
NJ = S // BK
GH = REP


def _attn_kern(q_ref, kt_ref, v_ref, o_ref, qp_scr, acc_scr, m_scr, l_scr):
    qb = pl.program_id(1)
    npair = qb + 1
    qp_scr[...] = jnp.concatenate(
        [q_ref[:, h * HD:(h + 1) * HD] for h in range(GH)], axis=0)
    row = (jax.lax.broadcasted_iota(jnp.int32, (GH * BQ, BK), 0) & (BQ - 1)) + qb * BQ
    col = jax.lax.broadcasted_iota(jnp.int32, (GH * BQ, BK), 1)
    qp = qp_scr[...]

    def body(t, _):
        ja = 2 * t
        jb = 2 * t + 1
        sa = jnp.dot(qp, kt_ref[0, ja], preferred_element_type=_f32)
        sb = jnp.dot(qp, kt_ref[0, jb], preferred_element_type=_f32)
        sa = jnp.where(col + ja * BK <= row, sa, _f32(-1e9))
        sb = jnp.where(col + jb * BK <= row, sb, _f32(-1e9))
        m_old = m_scr[...]
        m_new = jnp.maximum(
            m_old,
            jnp.maximum(jnp.max(sa, axis=-1, keepdims=True),
                        jnp.max(sb, axis=-1, keepdims=True)))
        pa = jnp.exp(sa - m_new)
        pb = jnp.exp(sb - m_new)
        pv = (jnp.dot(pa, v_ref[0, ja], preferred_element_type=_f32)
              + jnp.dot(pb, v_ref[0, jb], preferred_element_type=_f32))
        rs = (jnp.sum(pa, axis=-1, keepdims=True)
              + jnp.sum(pb, axis=-1, keepdims=True))
        corr = jnp.exp(m_old - m_new)

        @pl.when(t == 0)
        def _():
            acc_scr[...] = pv
            l_scr[...] = rs

        @pl.when(t > 0)
        def _():
            acc_scr[...] = acc_scr[...] * corr + pv
            l_scr[...] = l_scr[...] * corr + rs

        m_scr[...] = m_new
        return 0

    m_scr[...] = jnp.full((GH * BQ, 1), _f32(-1e30))
    jax.lax.fori_loop(0, npair, body, 0)
    accn = (acc_scr[...] / l_scr[...]).reshape(GH, BQ, HD)
    o_ref[...] = jnp.concatenate([accn[i] for i in range(GH)], axis=1)


def _proj_kern(attn_ref, wo_ref, res_ref, g2_ref, wr_ref, h_ref, x2_ref, lg_ref):
    hh = res_ref[...] + jnp.dot(attn_ref[...], wo_ref[...], preferred_element_type=_f32)
    h_ref[...] = hh
    mu = jnp.mean(hh, axis=-1, keepdims=True)
    var = jnp.mean((hh - mu) ** 2, axis=-1, keepdims=True)
    x2 = (hh - mu) * jax.lax.rsqrt(var + EPS) * g2_ref[...]
    x2_ref[...] = x2
    lg_ref[...] = jnp.dot(x2, wr_ref[...], preferred_element_type=_f32)


def _route_kern(lg_ref, gi1_ref, gi2_ref, w1_ref, w2_ref):
    lg = lg_ref[...]
    m = jnp.max(lg, axis=-1, keepdims=True)
    ex = jnp.exp(lg - m)
    p = ex / jnp.sum(ex, axis=-1, keepdims=True)
    lane = jax.lax.broadcasted_iota(jnp.int32, (S, E), 1)
    v1 = jnp.max(p, axis=-1, keepdims=True)
    i1 = jnp.min(jnp.where(p == v1, lane, E), axis=-1, keepdims=True)
    p2 = jnp.where(lane == i1, _f32(-1.0), p)
    v2 = jnp.max(p2, axis=-1, keepdims=True)
    i2 = jnp.min(jnp.where(p2 == v2, lane, E), axis=-1, keepdims=True)
    wsum = v1 + v2
    oh = (lane == i1).astype(_f32) + (lane == i2).astype(_f32)
    tri = (jax.lax.broadcasted_iota(jnp.int32, (S, S), 0)
           > jax.lax.broadcasted_iota(jnp.int32, (S, S), 1)).astype(_f32)
    cex = jnp.dot(tri, oh, preferred_element_type=_f32)
    pos1 = jnp.sum(jnp.where(lane == i1, cex, 0.0), axis=-1, keepdims=True).astype(jnp.int32)
    pos2 = jnp.sum(jnp.where(lane == i2, cex, 0.0), axis=-1, keepdims=True).astype(jnp.int32)
    keep1 = pos1 < C
    keep2 = pos2 < C
    gi1_ref[...] = jnp.where(keep1, i1 * C + pos1, EC)
    gi2_ref[...] = jnp.where(keep2, i2 * C + pos2, EC)
    w1_ref[...] = jnp.where(keep1, v1 / wsum, 0.0)
    w2_ref[...] = jnp.where(keep2, v2 / wsum, 0.0)


NW = 32
TPW = S // NW


def _sc_dispatch(x2_hbm, gi1_hbm, gi2_hbm, buf_hbm, rows_v, idx_v, sem):
    wid = jax.lax.axis_index("s") * 2 + jax.lax.axis_index("c")
    base = wid * TPW
    pltpu.sync_copy(x2_hbm.at[pl.ds(base, TPW)], rows_v)
    pltpu.sync_copy(gi1_hbm.at[pl.ds(base, TPW)], idx_v)
    pltpu.async_copy(rows_v, buf_hbm.at[idx_v], sem).wait()
    pltpu.sync_copy(gi2_hbm.at[pl.ds(base, TPW)], idx_v)
    pltpu.async_copy(rows_v, buf_hbm.at[idx_v], sem).wait()


def _sc_combine(oe_hbm, gi1_hbm, gi2_hbm, g1_hbm, g2_hbm, rows_v, idx_v, sem):
    wid = jax.lax.axis_index("s") * 2 + jax.lax.axis_index("c")
    base = wid * TPW
    pltpu.sync_copy(gi1_hbm.at[pl.ds(base, TPW)], idx_v)
    pltpu.async_copy(oe_hbm.at[idx_v], rows_v, sem).wait()
    pltpu.sync_copy(rows_v, g1_hbm.at[pl.ds(base, TPW)])
    pltpu.sync_copy(gi2_hbm.at[pl.ds(base, TPW)], idx_v)
    pltpu.async_copy(oe_hbm.at[idx_v], rows_v, sem).wait()
    pltpu.sync_copy(rows_v, g2_hbm.at[pl.ds(base, TPW)])


def _ffn_kern(buf_ref, wg_ref, wu_ref, wd_ref, o_ref):
    e = pl.program_id(0)
    f = pl.program_id(1)

    @pl.when(e < E)
    def _():
        b = buf_ref[...]
        a = jnp.dot(b, wg_ref[0], preferred_element_type=_f32)
        u = jnp.dot(b, wu_ref[0], preferred_element_type=_f32)
        g = a / (1.0 + jnp.exp(-a)) * u
        contrib = jnp.dot(g, wd_ref[0], preferred_element_type=_f32)

        @pl.when(f == 0)
        def _():
            o_ref[...] = contrib

        @pl.when(f > 0)
        def _():
            o_ref[...] += contrib

    @pl.when((e == E) & (f == 0))
    def _():
        o_ref[...] = jnp.zeros((C, D), _f32)


def _epilogue_kern(h_ref, g1_ref, g2_ref, w1_ref, w2_ref, o_ref):
    o_ref[...] = (h_ref[...] + g1_ref[...] * w1_ref[...]
                  + g2_ref[...] * w2_ref[...])


def kernel(hidden_states, attention_mask, position_ids, gamma1, gamma2,
           W_qkv, W_o, W_router, W_gate, W_up, W_down):
    del attention_mask
    x = hidden_states.reshape(S, D)
    pos = position_ids.reshape(S, 1)
    g1 = gamma1.reshape(1, D)
    g2 = gamma2.reshape(1, D)

    nq = S // BSQ
    q, kt, v3 = pl.pallas_call(
        _prologue_kern,
        grid=(nq,),
        in_specs=[
            pl.BlockSpec((BSQ, D), lambda i: (i, 0)),
            pl.BlockSpec((BSQ, 1), lambda i: (i, 0)),
            pl.BlockSpec((1, D), lambda i: (0, 0)),
            pl.BlockSpec((D, QKVW), lambda i: (0, 0)),
        ],
        out_specs=[
            pl.BlockSpec((BSQ, D), lambda i: (i, 0)),
            pl.BlockSpec((KV * HD, BSQ), lambda i: (0, i)),
            pl.BlockSpec((KV, BSQ, HD), lambda i: (0, i, 0)),
        ],
        out_shape=[
            jax.ShapeDtypeStruct((S, D), _f32),
            jax.ShapeDtypeStruct((KV * HD, S), _f32),
            jax.ShapeDtypeStruct((KV, S, HD), _f32),
        ],
    )(x, pos, g1, W_qkv)

    kt4 = kt.reshape(KV, HD, NJ, BK).transpose(0, 2, 1, 3)
    v4 = v3.reshape(KV, NJ, BK, HD)
    attn = pl.pallas_call(
        _attn_kern,
        grid=(KV, NQB),
        in_specs=[
            pl.BlockSpec((BQ, GH * HD), lambda g, i: (i, g)),
            pl.BlockSpec((1, NJ, HD, BK), lambda g, i: (g, 0, 0, 0)),
            pl.BlockSpec((1, NJ, BK, HD), lambda g, i: (g, 0, 0, 0)),
        ],
        out_specs=pl.BlockSpec((BQ, GH * HD), lambda g, i: (i, g)),
        out_shape=jax.ShapeDtypeStruct((S, D), _f32),
        scratch_shapes=[
            pltpu.VMEM((GH * BQ, HD), _f32),
            pltpu.VMEM((GH * BQ, HD), _f32),
            pltpu.VMEM((GH * BQ, 1), _f32),
            pltpu.VMEM((GH * BQ, 1), _f32),
        ],
    )(q, kt4, v4)

    h, x2, logits = pl.pallas_call(
        _proj_kern,
        grid=(nq,),
        in_specs=[
            pl.BlockSpec((BSQ, D), lambda i: (i, 0)),
            pl.BlockSpec((D, D), lambda i: (0, 0)),
            pl.BlockSpec((BSQ, D), lambda i: (i, 0)),
            pl.BlockSpec((1, D), lambda i: (0, 0)),
            pl.BlockSpec((D, E), lambda i: (0, 0)),
        ],
        out_specs=[
            pl.BlockSpec((BSQ, D), lambda i: (i, 0)),
            pl.BlockSpec((BSQ, D), lambda i: (i, 0)),
            pl.BlockSpec((BSQ, E), lambda i: (i, 0)),
        ],
        out_shape=[
            jax.ShapeDtypeStruct((S, D), _f32),
            jax.ShapeDtypeStruct((S, D), _f32),
            jax.ShapeDtypeStruct((S, E), _f32),
        ],
    )(attn, W_o, x, g2, W_router)

    gi1, gi2, w1, w2 = pl.pallas_call(
        _route_kern,
        grid=(1,),
        in_specs=[pl.BlockSpec((S, E), lambda i: (0, 0))],
        out_specs=[
            pl.BlockSpec((S, 1), lambda i: (0, 0)),
            pl.BlockSpec((S, 1), lambda i: (0, 0)),
            pl.BlockSpec((S, 1), lambda i: (0, 0)),
            pl.BlockSpec((S, 1), lambda i: (0, 0)),
        ],
        out_shape=[
            jax.ShapeDtypeStruct((S, 1), jnp.int32),
            jax.ShapeDtypeStruct((S, 1), jnp.int32),
            jax.ShapeDtypeStruct((S, 1), _f32),
            jax.ShapeDtypeStruct((S, 1), _f32),
        ],
    )(logits)

    gi1f = gi1.reshape(S)
    gi2f = gi2.reshape(S)
    mesh = plsc.VectorSubcoreMesh(core_axis_name="c", subcore_axis_name="s")

    buf = pl.kernel(
        _sc_dispatch,
        out_type=jax.ShapeDtypeStruct((EC + C, D), _f32),
        mesh=mesh,
        scratch_types=[
            pltpu.VMEM((TPW, D), _f32),
            pltpu.VMEM((TPW,), jnp.int32),
            pltpu.SemaphoreType.DMA,
        ],
    )(x2, gi1f, gi2f)

    oe = pl.pallas_call(
        _ffn_kern,
        grid=(E + 1, DFF // FB),
        in_specs=[
            pl.BlockSpec((C, D), lambda e, f: (e, 0)),
            pl.BlockSpec((1, D, FB), lambda e, f: (jnp.minimum(e, E - 1), 0, f)),
            pl.BlockSpec((1, D, FB), lambda e, f: (jnp.minimum(e, E - 1), 0, f)),
            pl.BlockSpec((1, FB, D), lambda e, f: (jnp.minimum(e, E - 1), f, 0)),
        ],
        out_specs=pl.BlockSpec((C, D), lambda e, f: (e, 0)),
        out_shape=jax.ShapeDtypeStruct((EC + C, D), _f32),
    )(buf, W_gate, W_up, W_down)

    g1, g2 = pl.kernel(
        _sc_combine,
        out_type=[
            jax.ShapeDtypeStruct((S, D), _f32),
            jax.ShapeDtypeStruct((S, D), _f32),
        ],
        mesh=mesh,
        scratch_types=[
            pltpu.VMEM((TPW, D), _f32),
            pltpu.VMEM((TPW,), jnp.int32),
            pltpu.SemaphoreType.DMA,
        ],
    )(oe, gi1f, gi2f)

    out = pl.pallas_call(
        _epilogue_kern,
        grid=(nq,),
        in_specs=[
            pl.BlockSpec((BSQ, D), lambda i: (i, 0)),
            pl.BlockSpec((BSQ, D), lambda i: (i, 0)),
            pl.BlockSpec((BSQ, D), lambda i: (i, 0)),
            pl.BlockSpec((BSQ, 1), lambda i: (i, 0)),
            pl.BlockSpec((BSQ, 1), lambda i: (i, 0)),
        ],
        out_specs=pl.BlockSpec((BSQ, D), lambda i: (i, 0)),
        out_shape=jax.ShapeDtypeStruct((S, D), _f32),
    )(h, g1, g2, w1, w2)

    return out.reshape(B, S, D)
